# baseline (speedup 1.0000x reference)
import functools

import jax
import jax.numpy as jnp
from jax import lax
from jax.experimental import pallas as pl
from jax.experimental.pallas import tpu as pltpu
from jax.experimental.pallas import tpu_sc as plsc

B = 16384
D = 128
H = 128
NC = 2
NS = 16
NW = NC * NS
BPW = B // NW
C = 256
NCHUNK = BPW // C

def _gather_mul_body(uid_hbm, iid_hbm, utab_hbm, itab_hbm, out_hbm,
                     uidx, iidx, urows, irows, sem):
    wid = lax.axis_index("s") * NC + lax.axis_index("c")
    base = wid * BPW
    for c in range(NCHUNK):
        off = base + c * C
        pltpu.sync_copy(uid_hbm.at[pl.ds(off, C)], uidx)
        pltpu.sync_copy(iid_hbm.at[pl.ds(off, C)], iidx)
        cu = pltpu.make_async_copy(utab_hbm.at[uidx], urows, sem)
        ci = pltpu.make_async_copy(itab_hbm.at[iidx], irows, sem)
        cu.start()
        ci.start()
        cu.wait()
        ci.wait()

        def mul_row(i, carry):
            for j in range(D // 16):
                sl = pl.ds(j * 16, 16)
                urows[i, sl] = urows[i, sl] * irows[i, sl]
            return carry

        lax.fori_loop(0, C, mul_row, 0)
        pltpu.sync_copy(urows, out_hbm.at[pl.ds(off, C)])


@functools.cache
def _gather_mul_fn():
    mesh = plsc.VectorSubcoreMesh(core_axis_name="c", subcore_axis_name="s")
    return pl.kernel(
        _gather_mul_body,
        mesh=mesh,
        out_type=jax.ShapeDtypeStruct((B, D), jnp.float32),
        scratch_types=[
            pltpu.VMEM((C,), jnp.int32),
            pltpu.VMEM((C,), jnp.int32),
            pltpu.VMEM((C, D), jnp.float32),
            pltpu.VMEM((C, D), jnp.float32),
            pltpu.SemaphoreType.DMA,
        ],
    )


def _mlp_body(x_ref, w1_ref, b1_ref, w2_ref, b2_ref, o_ref):
    x = x_ref[...]
    h = jnp.dot(x, w1_ref[...], preferred_element_type=jnp.float32) + b1_ref[...]
    h = jnp.maximum(h, 0.0)
    z = jnp.dot(h, w2_ref[...], preferred_element_type=jnp.float32) + b2_ref[...]
    o_ref[...] = 1.0 / (1.0 + jnp.exp(-z))


def _mlp(x, W1, b1, W2, b2):
    BT = 2048
    return pl.pallas_call(
        _mlp_body,
        grid=(B // BT,),
        in_specs=[
            pl.BlockSpec((BT, D), lambda i: (i, 0)),
            pl.BlockSpec((D, H), lambda i: (0, 0)),
            pl.BlockSpec((1, H), lambda i: (0, 0)),
            pl.BlockSpec((H, 1), lambda i: (0, 0)),
            pl.BlockSpec((1, 1), lambda i: (0, 0)),
        ],
        out_specs=pl.BlockSpec((BT, 1), lambda i: (i, 0)),
        out_shape=jax.ShapeDtypeStruct((B, 1), jnp.float32),
    )(x, W1, b1.reshape(1, H), W2, b2.reshape(1, 1))


def kernel(user_id, item_id, user_table, item_table, W1, b1, W2, b2):
    uid = user_id.reshape(B).astype(jnp.int32)
    iid = item_id.reshape(B).astype(jnp.int32)
    x = _gather_mul_fn()(uid, iid, user_table, item_table)
    out = _mlp(x, W1, b1, W2, b2)
    return out.reshape(B, 1, 1)


# speedup vs baseline: 1.0834x; 1.0834x over previous
import functools

import jax
import jax.numpy as jnp
from jax import lax
from jax.experimental import pallas as pl
from jax.experimental.pallas import tpu as pltpu
from jax.experimental.pallas import tpu_sc as plsc

B = 16384
D = 128
H = 128
NC = 2
NS = 16
NW = NC * NS
BPW = B // NW
C = 128
NCHUNK = BPW // C

def _gather_mul_body(uid_hbm, iid_hbm, utab_hbm, itab_hbm, out_hbm,
                     uidx, iidx, u0, u1, i0, i1,
                     sem_idx, sem_g0, sem_g1, sem_w0, sem_w1):
    wid = lax.axis_index("s") * NC + lax.axis_index("c")
    base = wid * BPW
    ubufs = (u0, u1)
    ibufs = (i0, i1)
    gsems = (sem_g0, sem_g1)
    wsems = (sem_w0, sem_w1)

    cu_idx = pltpu.make_async_copy(uid_hbm.at[pl.ds(base, BPW)], uidx, sem_idx)
    ci_idx = pltpu.make_async_copy(iid_hbm.at[pl.ds(base, BPW)], iidx, sem_idx)
    cu_idx.start()
    ci_idx.start()
    cu_idx.wait()
    ci_idx.wait()

    def fire_gather(c):
        s = c % 2
        g_u = pltpu.make_async_copy(
            utab_hbm.at[uidx.at[pl.ds(c * C, C)]], ubufs[s], gsems[s])
        g_i = pltpu.make_async_copy(
            itab_hbm.at[iidx.at[pl.ds(c * C, C)]], ibufs[s], gsems[s])
        g_u.start()
        g_i.start()
        return g_u, g_i

    writes = [None, None]
    pend = fire_gather(0)
    for c in range(NCHUNK):
        s = c % 2
        if c + 1 < NCHUNK:
            if writes[(c + 1) % 2] is not None:
                writes[(c + 1) % 2].wait()
                writes[(c + 1) % 2] = None
            nxt = fire_gather(c + 1)
        pend[0].wait()
        pend[1].wait()
        if c + 1 < NCHUNK:
            pend = nxt

        urows, irows = ubufs[s], ibufs[s]

        def mul_row(i, carry):
            for j in range(D // 16):
                sl = pl.ds(j * 16, 16)
                urows[i, sl] = urows[i, sl] * irows[i, sl]
            return carry

        lax.fori_loop(0, C, mul_row, 0)
        w = pltpu.make_async_copy(
            urows, out_hbm.at[pl.ds(base + c * C, C)], wsems[s])
        w.start()
        writes[s] = w
    for w in writes:
        if w is not None:
            w.wait()


@functools.cache
def _gather_mul_fn():
    mesh = plsc.VectorSubcoreMesh(core_axis_name="c", subcore_axis_name="s")
    return pl.kernel(
        _gather_mul_body,
        mesh=mesh,
        out_type=jax.ShapeDtypeStruct((B, D), jnp.float32),
        scratch_types=[
            pltpu.VMEM((BPW,), jnp.int32),
            pltpu.VMEM((BPW,), jnp.int32),
            pltpu.VMEM((C, D), jnp.float32),
            pltpu.VMEM((C, D), jnp.float32),
            pltpu.VMEM((C, D), jnp.float32),
            pltpu.VMEM((C, D), jnp.float32),
            pltpu.SemaphoreType.DMA,
            pltpu.SemaphoreType.DMA,
            pltpu.SemaphoreType.DMA,
            pltpu.SemaphoreType.DMA,
            pltpu.SemaphoreType.DMA,
        ],
    )


def _mlp_body(x_ref, w1_ref, b1_ref, w2_ref, b2_ref, o_ref):
    x = x_ref[...]
    h = jnp.dot(x, w1_ref[...], preferred_element_type=jnp.float32) + b1_ref[...]
    h = jnp.maximum(h, 0.0)
    z = jnp.dot(h, w2_ref[...], preferred_element_type=jnp.float32) + b2_ref[...]
    o_ref[...] = 1.0 / (1.0 + jnp.exp(-z))


def _mlp(x, W1, b1, W2, b2):
    BT = 2048
    return pl.pallas_call(
        _mlp_body,
        grid=(B // BT,),
        in_specs=[
            pl.BlockSpec((BT, D), lambda i: (i, 0)),
            pl.BlockSpec((D, H), lambda i: (0, 0)),
            pl.BlockSpec((1, H), lambda i: (0, 0)),
            pl.BlockSpec((H, 1), lambda i: (0, 0)),
            pl.BlockSpec((1, 1), lambda i: (0, 0)),
        ],
        out_specs=pl.BlockSpec((BT, 1), lambda i: (i, 0)),
        out_shape=jax.ShapeDtypeStruct((B, 1), jnp.float32),
    )(x, W1, b1.reshape(1, H), W2, b2.reshape(1, 1))


def kernel(user_id, item_id, user_table, item_table, W1, b1, W2, b2):
    uid = user_id.reshape(B).astype(jnp.int32)
    iid = item_id.reshape(B).astype(jnp.int32)
    x = _gather_mul_fn()(uid, iid, user_table, item_table)
    out = _mlp(x, W1, b1, W2, b2)
    return out.reshape(B, 1, 1)


# speedup vs baseline: 1.2952x; 1.1954x over previous
import functools

import jax
import jax.numpy as jnp
from jax import lax
from jax.experimental import pallas as pl
from jax.experimental.pallas import tpu as pltpu
from jax.experimental.pallas import tpu_sc as plsc

B = 16384
D = 128
H = 128
NC = 2
NS = 16
NW = NC * NS
BPW = B // NW
C = 128
NCHUNK = BPW // C

def _gather_mul_body(uid_hbm, iid_hbm, utab_hbm, itab_hbm, out_hbm,
                     uidx, iidx, u0, u1, i0, i1,
                     sem_idx, sem_g0, sem_g1, sem_w0, sem_w1):
    wid = lax.axis_index("s") * NC + lax.axis_index("c")
    base = wid * BPW
    ubufs = (u0, u1)
    ibufs = (i0, i1)
    gsems = (sem_g0, sem_g1)
    wsems = (sem_w0, sem_w1)

    cu_idx = pltpu.make_async_copy(uid_hbm.at[pl.ds(base, BPW)], uidx, sem_idx)
    ci_idx = pltpu.make_async_copy(iid_hbm.at[pl.ds(base, BPW)], iidx, sem_idx)
    cu_idx.start()
    ci_idx.start()
    cu_idx.wait()
    ci_idx.wait()

    def fire_gather(c):
        s = c % 2
        g_u = pltpu.make_async_copy(
            utab_hbm.at[uidx.at[pl.ds(c * C, C)]], ubufs[s], gsems[s])
        g_i = pltpu.make_async_copy(
            itab_hbm.at[iidx.at[pl.ds(c * C, C)]], ibufs[s], gsems[s])
        g_u.start()
        g_i.start()
        return g_u, g_i

    writes = [None, None]
    pend = fire_gather(0)
    for c in range(NCHUNK):
        s = c % 2
        if c + 1 < NCHUNK:
            if writes[(c + 1) % 2] is not None:
                writes[(c + 1) % 2].wait()
                writes[(c + 1) % 2] = None
            nxt = fire_gather(c + 1)
        pend[0].wait()
        pend[1].wait()
        if c + 1 < NCHUNK:
            pend = nxt

        urows, irows = ubufs[s], ibufs[s]

        def mul_row(i, carry):
            for j in range(D // 16):
                sl = pl.ds(j * 16, 16)
                urows[i, sl] = urows[i, sl] * irows[i, sl]
            return carry

        lax.fori_loop(0, C, mul_row, 0)
        w = pltpu.make_async_copy(
            urows, out_hbm.at[pl.ds(base + c * C, C)], wsems[s])
        w.start()
        writes[s] = w
    for w in writes:
        if w is not None:
            w.wait()


@functools.cache
def _gather_mul_fn():
    mesh = plsc.VectorSubcoreMesh(core_axis_name="c", subcore_axis_name="s")
    return pl.kernel(
        _gather_mul_body,
        mesh=mesh,
        out_type=jax.ShapeDtypeStruct((B, D), jnp.float32),
        scratch_types=[
            pltpu.VMEM((BPW,), jnp.int32),
            pltpu.VMEM((BPW,), jnp.int32),
            pltpu.VMEM((C, D), jnp.float32),
            pltpu.VMEM((C, D), jnp.float32),
            pltpu.VMEM((C, D), jnp.float32),
            pltpu.VMEM((C, D), jnp.float32),
            pltpu.SemaphoreType.DMA,
            pltpu.SemaphoreType.DMA,
            pltpu.SemaphoreType.DMA,
            pltpu.SemaphoreType.DMA,
            pltpu.SemaphoreType.DMA,
        ],
    )


def _mlp_body(x_ref, w1t_ref, b1_ref, w2_ref, b2_ref, o_ref):
    xt = x_ref[...].T
    ht = jnp.dot(w1t_ref[...], xt, preferred_element_type=jnp.float32)
    ht = jnp.maximum(ht + b1_ref[...], 0.0)
    z = jnp.dot(w2_ref[...], ht, preferred_element_type=jnp.float32) + b2_ref[0]
    o_ref[...] = 1.0 / (1.0 + jnp.exp(-z))


def _mlp(x, W1, b1, W2, b2):
    BT = 2048
    return pl.pallas_call(
        _mlp_body,
        grid=(B // BT,),
        in_specs=[
            pl.BlockSpec((BT, D), lambda i: (i, 0)),
            pl.BlockSpec((D, H), lambda i: (0, 0)),
            pl.BlockSpec((H, 1), lambda i: (0, 0)),
            pl.BlockSpec((1, H), lambda i: (0, 0)),
            pl.BlockSpec(memory_space=pltpu.SMEM),
        ],
        out_specs=pl.BlockSpec((1, BT), lambda i: (0, i)),
        out_shape=jax.ShapeDtypeStruct((1, B), jnp.float32),
    )(x, W1.T, b1.reshape(H, 1), W2.reshape(1, H), b2)


def kernel(user_id, item_id, user_table, item_table, W1, b1, W2, b2):
    uid = user_id.reshape(B).astype(jnp.int32)
    iid = item_id.reshape(B).astype(jnp.int32)
    x = _gather_mul_fn()(uid, iid, user_table, item_table)
    out = _mlp(x, W1, b1, W2, b2)
    return out.reshape(B, 1, 1)
